# baseline (speedup 1.0000x reference)
import functools

import jax
import jax.numpy as jnp
from jax import lax
from jax.experimental import pallas as pl
from jax.experimental.pallas import tpu as pltpu
from jax.experimental.pallas import tpu_sc as plsc

N = 10000
E = 320000
D = 128
H = 128
C = 64

NC = 2
NS = 16
NW = NC * NS

CHUNK = 128
CPW = 80
EPAD = NW * CPW * CHUNK
NPAD = 10240
RPT = NPAD // NS


def _seg_body(src_hbm, dst_hbm, x_hbm, zs_hbm, zc_hbm, ones_hbm,
              osum_hbm, ocnt0_hbm, ocnt1_hbm,
              acc_sum, acc_cnt, src_v, dst_v, rows0_v, ones_v,
              sem0, sem1):
    cid = lax.axis_index("c")
    sid = lax.axis_index("s")
    w = sid * NC + cid

    czs = pltpu.async_copy(zs_hbm, acc_sum.at[pl.ds(sid * RPT, RPT)], sem1)
    pltpu.sync_copy(zc_hbm, acc_cnt.at[pl.ds(sid * RPT, RPT)])
    pltpu.sync_copy(ones_hbm, ones_v)
    czs.wait()
    plsc.subcore_barrier()

    base0 = w * (CPW * CHUNK)

    @pl.loop(0, CPW)
    def _(j):
        base = pl.multiple_of(base0 + j * CHUNK, CHUNK)
        pltpu.sync_copy(src_hbm.at[pl.ds(base, CHUNK)], src_v)
        pltpu.sync_copy(dst_hbm.at[pl.ds(base, CHUNK)], dst_v)
        pltpu.async_copy(x_hbm.at[src_v], rows0_v, sem0).wait()
        pltpu.sync_copy(rows0_v, acc_sum.at[dst_v], add=True)
        pltpu.sync_copy(ones_v, acc_cnt.at[dst_v], add=True)

    plsc.subcore_barrier()
    pltpu.sync_copy(acc_sum.at[pl.ds(sid * RPT, RPT)],
                    osum_hbm.at[cid, pl.ds(sid * RPT, RPT)])
    @pl.when(cid == 0)
    def _():
        pltpu.sync_copy(acc_cnt.at[pl.ds(sid * RPT, RPT)],
                        ocnt0_hbm.at[pl.ds(sid * RPT, RPT)])

    @pl.when(cid == 1)
    def _():
        pltpu.sync_copy(acc_cnt.at[pl.ds(sid * RPT, RPT)],
                        ocnt1_hbm.at[pl.ds(sid * RPT, RPT)])


@functools.cache
def _make_seg_call():
    return pl.kernel(
        _seg_body,
        out_type=[
            jax.ShapeDtypeStruct((NC, NPAD, D), jnp.float32),
            jax.ShapeDtypeStruct((NPAD,), jnp.float32),
            jax.ShapeDtypeStruct((NPAD,), jnp.float32),
        ],
        mesh=plsc.VectorSubcoreMesh(core_axis_name="c", subcore_axis_name="s",
                                    num_cores=NC, num_subcores=NS),
        scratch_types=[
            pltpu.VMEM_SHARED((NPAD, D), jnp.float32),
            pltpu.VMEM_SHARED((NPAD,), jnp.float32),
            pltpu.VMEM((CHUNK,), jnp.int32),
            pltpu.VMEM((CHUNK,), jnp.int32),
            pltpu.VMEM((CHUNK, D), jnp.float32),
            pltpu.VMEM((CHUNK,), jnp.float32),
            pltpu.SemaphoreType.DMA,
            pltpu.SemaphoreType.DMA,
        ],
    )


BLK = 1000
NBLK = N // BLK


def _mmT(a, b):
    return lax.dot_general(a, b, (((1,), (1,)), ((), ())),
                           preferred_element_type=jnp.float32)


def _head_body(x_ref, ps_ref, pc0_ref, pc1_ref, wl_ref, bl_ref, wr_ref,
               w1_ref, b1_ref, w2_ref, b2_ref, o_ref, acc_ref):
    i = pl.program_id(0)

    @pl.when(i == 0)
    def _():
        acc_ref[...] = jnp.zeros_like(acc_ref)

    s = ps_ref[0] + ps_ref[1]
    cnt = pc0_ref[...] + pc1_ref[...]
    mean = s / jnp.maximum(cnt, 1.0)
    pre = _mmT(mean, wl_ref[...]) + bl_ref[...] + _mmT(x_ref[...], wr_ref[...])
    h = jnp.maximum(pre, 0.0)
    acc_ref[...] += jnp.sum(h, axis=0, keepdims=True)

    @pl.when(i == NBLK - 1)
    def _():
        pooled = acc_ref[...] * (1.0 / N)
        z = jnp.maximum(_mmT(pooled, w1_ref[...]) + b1_ref[...], 0.0)
        z2 = _mmT(z, w2_ref[...]) + b2_ref[...]
        m = jnp.max(z2, axis=1, keepdims=True)
        e = z2 - m
        lse = jnp.log(jnp.sum(jnp.exp(e), axis=1, keepdims=True))
        o_ref[...] = e - lse


@functools.cache
def _make_head_call(interpret: bool = False):
    return pl.pallas_call(
        _head_body,
        grid=(NBLK,),
        in_specs=[
            pl.BlockSpec((BLK, D), lambda i: (i, 0)),
            pl.BlockSpec((NC, BLK, D), lambda i: (0, i, 0)),
            pl.BlockSpec((BLK, 1), lambda i: (i, 0)),
            pl.BlockSpec((BLK, 1), lambda i: (i, 0)),
            pl.BlockSpec((H, D), lambda i: (0, 0)),
            pl.BlockSpec((1, H), lambda i: (0, 0)),
            pl.BlockSpec((H, D), lambda i: (0, 0)),
            pl.BlockSpec((H, H), lambda i: (0, 0)),
            pl.BlockSpec((1, H), lambda i: (0, 0)),
            pl.BlockSpec((C, H), lambda i: (0, 0)),
            pl.BlockSpec((1, C), lambda i: (0, 0)),
        ],
        out_specs=pl.BlockSpec((1, C), lambda i: (0, 0)),
        out_shape=jax.ShapeDtypeStruct((1, C), jnp.float32),
        scratch_shapes=[pltpu.VMEM((1, H), jnp.float32)],
        interpret=interpret,
    )


def kernel(x, edge_index, Wl0, bl0, Wr0, Wl1, bl1, Wr1, Wl2, bl2, Wr2,
           W1, b1, W2, b2):
    src = edge_index[0]
    dst = edge_index[1]
    pad = EPAD - E
    srcp = jnp.concatenate([src, jnp.zeros((pad,), jnp.int32)])
    dstp = jnp.concatenate([dst, jnp.full((pad,), N, jnp.int32)])
    zs = jnp.zeros((RPT, D), jnp.float32)
    zc = jnp.zeros((RPT,), jnp.float32)
    ones = jnp.ones((CHUNK,), jnp.float32)
    psum, cnt0, cnt1 = _make_seg_call()(srcp, dstp, x, zs, zc, ones)
    return _make_head_call()(x, psum, cnt0[:, None], cnt1[:, None], Wl2, bl2.reshape(1, H), Wr2,
                             W1, b1.reshape(1, H), W2, b2.reshape(1, C))


# speedup vs baseline: 2.1312x; 2.1312x over previous
import functools

import jax
import jax.numpy as jnp
from jax import lax
from jax.experimental import pallas as pl
from jax.experimental.pallas import tpu as pltpu
from jax.experimental.pallas import tpu_sc as plsc

N = 10000
E = 320000
D = 128
H = 128
C = 64

NC = 2
NS = 16
NW = NC * NS

CHUNK = 128
CPW = 79
EPAD = NW * CPW * CHUNK
NPAD = 10240
RPT = NPAD // NS


def _seg_body(src_hbm, dst_hbm, x_hbm, zs_hbm, zc_hbm, ones_hbm,
              osum_hbm, ocnt0_hbm, ocnt1_hbm,
              acc_sum, acc_cnt, src_v, dst_v, rows0_v, ones_v,
              sem0, sem1):
    cid = lax.axis_index("c")
    sid = lax.axis_index("s")
    w = sid * NC + cid

    czs = pltpu.async_copy(zs_hbm, acc_sum.at[pl.ds(sid * RPT, RPT)], sem1)
    pltpu.sync_copy(zc_hbm, acc_cnt.at[pl.ds(sid * RPT, RPT)])
    pltpu.sync_copy(ones_hbm, ones_v)
    czs.wait()
    plsc.subcore_barrier()

    base0 = w * (CPW * CHUNK)

    @pl.loop(0, CPW)
    def _(j):
        base = pl.multiple_of(base0 + j * CHUNK, CHUNK)
        pltpu.sync_copy(src_hbm.at[pl.ds(base, CHUNK)], src_v)
        pltpu.sync_copy(dst_hbm.at[pl.ds(base, CHUNK)], dst_v)
        pltpu.async_copy(x_hbm.at[src_v], rows0_v, sem0).wait()
        pltpu.sync_copy(rows0_v, acc_sum.at[dst_v], add=True)
        pltpu.sync_copy(ones_v, acc_cnt.at[dst_v], add=True)

    plsc.subcore_barrier()
    pltpu.sync_copy(acc_sum.at[pl.ds(sid * RPT, RPT)],
                    osum_hbm.at[cid, pl.ds(sid * RPT, RPT)])
    @pl.when(cid == 0)
    def _():
        pltpu.sync_copy(acc_cnt.at[pl.ds(sid * RPT, RPT)],
                        ocnt0_hbm.at[pl.ds(sid * RPT, RPT)])

    @pl.when(cid == 1)
    def _():
        pltpu.sync_copy(acc_cnt.at[pl.ds(sid * RPT, RPT)],
                        ocnt1_hbm.at[pl.ds(sid * RPT, RPT)])


@functools.cache
def _make_seg_call():
    return pl.kernel(
        _seg_body,
        out_type=[
            jax.ShapeDtypeStruct((NC, NPAD, D), jnp.float32),
            jax.ShapeDtypeStruct((NPAD,), jnp.float32),
            jax.ShapeDtypeStruct((NPAD,), jnp.float32),
        ],
        mesh=plsc.VectorSubcoreMesh(core_axis_name="c", subcore_axis_name="s",
                                    num_cores=NC, num_subcores=NS),
        scratch_types=[
            pltpu.VMEM_SHARED((NPAD, D), jnp.float32),
            pltpu.VMEM_SHARED((NPAD,), jnp.float32),
            pltpu.VMEM((CHUNK,), jnp.int32),
            pltpu.VMEM((CHUNK,), jnp.int32),
            pltpu.VMEM((CHUNK, D), jnp.float32),
            pltpu.VMEM((CHUNK,), jnp.float32),
            pltpu.SemaphoreType.DMA,
            pltpu.SemaphoreType.DMA,
        ],
    )


BLK = 1000
NBLK = N // BLK


def _mmT(a, b):
    return lax.dot_general(a, b, (((1,), (1,)), ((), ())),
                           preferred_element_type=jnp.float32)


def _head_body(x_ref, ps_ref, pc0_ref, pc1_ref, wl_ref, bl_ref, wr_ref,
               w1_ref, b1_ref, w2_ref, b2_ref, o_ref, acc_ref):
    i = pl.program_id(0)

    @pl.when(i == 0)
    def _():
        acc_ref[...] = jnp.zeros_like(acc_ref)

    s = ps_ref[0] + ps_ref[1]
    cnt = pc0_ref[...] + pc1_ref[...]
    mean = s / jnp.maximum(cnt, 1.0)
    pre = _mmT(mean, wl_ref[...]) + bl_ref[...] + _mmT(x_ref[...], wr_ref[...])
    h = jnp.maximum(pre, 0.0)
    acc_ref[...] += jnp.sum(h, axis=0, keepdims=True)

    @pl.when(i == NBLK - 1)
    def _():
        pooled = acc_ref[...] * (1.0 / N)
        z = jnp.maximum(_mmT(pooled, w1_ref[...]) + b1_ref[...], 0.0)
        z2 = _mmT(z, w2_ref[...]) + b2_ref[...]
        m = jnp.max(z2, axis=1, keepdims=True)
        e = z2 - m
        lse = jnp.log(jnp.sum(jnp.exp(e), axis=1, keepdims=True))
        o_ref[...] = e - lse


@functools.cache
def _make_head_call(interpret: bool = False):
    return pl.pallas_call(
        _head_body,
        grid=(NBLK,),
        in_specs=[
            pl.BlockSpec((BLK, D), lambda i: (i, 0)),
            pl.BlockSpec((NC, BLK, D), lambda i: (0, i, 0)),
            pl.BlockSpec((BLK, 1), lambda i: (i, 0)),
            pl.BlockSpec((BLK, 1), lambda i: (i, 0)),
            pl.BlockSpec((H, D), lambda i: (0, 0)),
            pl.BlockSpec((1, H), lambda i: (0, 0)),
            pl.BlockSpec((H, D), lambda i: (0, 0)),
            pl.BlockSpec((H, H), lambda i: (0, 0)),
            pl.BlockSpec((1, H), lambda i: (0, 0)),
            pl.BlockSpec((C, H), lambda i: (0, 0)),
            pl.BlockSpec((1, C), lambda i: (0, 0)),
        ],
        out_specs=pl.BlockSpec((1, C), lambda i: (0, 0)),
        out_shape=jax.ShapeDtypeStruct((1, C), jnp.float32),
        scratch_shapes=[pltpu.VMEM((1, H), jnp.float32)],
        interpret=interpret,
    )


def kernel(x, edge_index, Wl0, bl0, Wr0, Wl1, bl1, Wr1, Wl2, bl2, Wr2,
           W1, b1, W2, b2):
    src = edge_index[0]
    dst = edge_index[1]
    pad = EPAD - E
    ar = jnp.arange(pad, dtype=jnp.int32)
    srcp = jnp.concatenate([src, ar % N])
    dstp = jnp.concatenate([dst, N + ar % (NPAD - N)])
    zs = jnp.zeros((RPT, D), jnp.float32)
    zc = jnp.zeros((RPT,), jnp.float32)
    ones = jnp.ones((CHUNK,), jnp.float32)
    psum, cnt0, cnt1 = _make_seg_call()(srcp, dstp, x, zs, zc, ones)
    return _make_head_call()(x, psum, cnt0[:, None], cnt1[:, None], Wl2, bl2.reshape(1, H), Wr2,
                             W1, b1.reshape(1, H), W2, b2.reshape(1, C))


# speedup vs baseline: 3.8792x; 1.8202x over previous
import functools

import jax
import jax.numpy as jnp
from jax import lax
from jax.experimental import pallas as pl
from jax.experimental.pallas import tpu as pltpu
from jax.experimental.pallas import tpu_sc as plsc

N = 10000
E = 320000
D = 128
H = 128
C = 64

NC = 2
NS = 16
NW = NC * NS

CHUNK = 128
CPW = 80
HPW = 40
EPAD = NW * CPW * CHUNK
NPAD = 10240
RPT = NPAD // NS


def _seg_body(src_hbm, dst_hbm, x_hbm, zs_hbm, zc_hbm, ones_hbm,
              osum_hbm, ocnt0_hbm, ocnt1_hbm,
              acc_sum, acc_cnt, src_v, dst_v, rows0_v, rows1_v, ones_v,
              sem0, sem1):
    cid = lax.axis_index("c")
    sid = lax.axis_index("s")
    w = sid * NC + cid

    czs = pltpu.async_copy(zs_hbm, acc_sum.at[pl.ds(sid * RPT, RPT)], sem1)
    pltpu.sync_copy(zc_hbm, acc_cnt.at[pl.ds(sid * RPT, RPT)])
    pltpu.sync_copy(ones_hbm, ones_v)
    czs.wait()
    plsc.subcore_barrier()

    for h in range(2):
        pltpu.sync_copy(src_hbm.at[pl.ds(w * CPW + h * HPW, HPW)], src_v)
        pltpu.sync_copy(dst_hbm.at[pl.ds(w * CPW + h * HPW, HPW)], dst_v)
        pltpu.async_copy(x_hbm.at[src_v.at[0]], rows0_v, sem0)

        @pl.loop(0, HPW, step=2)
        def _(j0):
            j1 = j0 + 1
            pltpu.async_copy(x_hbm.at[src_v.at[j1]], rows1_v, sem1)
            pltpu.make_async_copy(x_hbm.at[src_v.at[j0]], rows0_v, sem0).wait()
            pltpu.sync_copy(rows0_v, acc_sum.at[dst_v.at[j0]], add=True)
            pltpu.sync_copy(ones_v, acc_cnt.at[dst_v.at[j0]], add=True)

            @pl.when(j0 + 2 < HPW)
            def _():
                pltpu.async_copy(x_hbm.at[src_v.at[j0 + 2]], rows0_v, sem0)

            pltpu.make_async_copy(x_hbm.at[src_v.at[j1]], rows1_v, sem1).wait()
            pltpu.sync_copy(rows1_v, acc_sum.at[dst_v.at[j1]], add=True)
            pltpu.sync_copy(ones_v, acc_cnt.at[dst_v.at[j1]], add=True)

    plsc.subcore_barrier()
    pltpu.sync_copy(acc_sum.at[pl.ds(sid * RPT, RPT)],
                    osum_hbm.at[cid, pl.ds(sid * RPT, RPT)])
    @pl.when(cid == 0)
    def _():
        pltpu.sync_copy(acc_cnt.at[pl.ds(sid * RPT, RPT)],
                        ocnt0_hbm.at[pl.ds(sid * RPT, RPT)])

    @pl.when(cid == 1)
    def _():
        pltpu.sync_copy(acc_cnt.at[pl.ds(sid * RPT, RPT)],
                        ocnt1_hbm.at[pl.ds(sid * RPT, RPT)])


@functools.cache
def _make_seg_call():
    return pl.kernel(
        _seg_body,
        out_type=[
            jax.ShapeDtypeStruct((NC, NPAD, D), jnp.float32),
            jax.ShapeDtypeStruct((NPAD,), jnp.float32),
            jax.ShapeDtypeStruct((NPAD,), jnp.float32),
        ],
        mesh=plsc.VectorSubcoreMesh(core_axis_name="c", subcore_axis_name="s",
                                    num_cores=NC, num_subcores=NS),
        scratch_types=[
            pltpu.VMEM_SHARED((NPAD, D), jnp.float32),
            pltpu.VMEM_SHARED((NPAD,), jnp.float32),
            pltpu.VMEM((HPW, CHUNK), jnp.int32),
            pltpu.VMEM((HPW, CHUNK), jnp.int32),
            pltpu.VMEM((CHUNK, D), jnp.float32),
            pltpu.VMEM((CHUNK, D), jnp.float32),
            pltpu.VMEM((CHUNK,), jnp.float32),
            pltpu.SemaphoreType.DMA,
            pltpu.SemaphoreType.DMA,
        ],
    )


BLK = 1000
NBLK = N // BLK


def _mmT(a, b):
    return lax.dot_general(a, b, (((1,), (1,)), ((), ())),
                           preferred_element_type=jnp.float32)


def _head_body(x_ref, ps_ref, pc0_ref, pc1_ref, wl_ref, bl_ref, wr_ref,
               w1_ref, b1_ref, w2_ref, b2_ref, o_ref, acc_ref):
    i = pl.program_id(0)

    @pl.when(i == 0)
    def _():
        acc_ref[...] = jnp.zeros_like(acc_ref)

    s = ps_ref[0] + ps_ref[1]
    cnt = pc0_ref[...] + pc1_ref[...]
    mean = s / jnp.maximum(cnt, 1.0)
    pre = _mmT(mean, wl_ref[...]) + bl_ref[...] + _mmT(x_ref[...], wr_ref[...])
    h = jnp.maximum(pre, 0.0)
    acc_ref[...] += jnp.sum(h, axis=0, keepdims=True)

    @pl.when(i == NBLK - 1)
    def _():
        pooled = acc_ref[...] * (1.0 / N)
        z = jnp.maximum(_mmT(pooled, w1_ref[...]) + b1_ref[...], 0.0)
        z2 = _mmT(z, w2_ref[...]) + b2_ref[...]
        m = jnp.max(z2, axis=1, keepdims=True)
        e = z2 - m
        lse = jnp.log(jnp.sum(jnp.exp(e), axis=1, keepdims=True))
        o_ref[...] = e - lse


@functools.cache
def _make_head_call(interpret: bool = False):
    return pl.pallas_call(
        _head_body,
        grid=(NBLK,),
        in_specs=[
            pl.BlockSpec((BLK, D), lambda i: (i, 0)),
            pl.BlockSpec((NC, BLK, D), lambda i: (0, i, 0)),
            pl.BlockSpec((BLK, 1), lambda i: (i, 0)),
            pl.BlockSpec((BLK, 1), lambda i: (i, 0)),
            pl.BlockSpec((H, D), lambda i: (0, 0)),
            pl.BlockSpec((1, H), lambda i: (0, 0)),
            pl.BlockSpec((H, D), lambda i: (0, 0)),
            pl.BlockSpec((H, H), lambda i: (0, 0)),
            pl.BlockSpec((1, H), lambda i: (0, 0)),
            pl.BlockSpec((C, H), lambda i: (0, 0)),
            pl.BlockSpec((1, C), lambda i: (0, 0)),
        ],
        out_specs=pl.BlockSpec((1, C), lambda i: (0, 0)),
        out_shape=jax.ShapeDtypeStruct((1, C), jnp.float32),
        scratch_shapes=[pltpu.VMEM((1, H), jnp.float32)],
        interpret=interpret,
    )


def kernel(x, edge_index, Wl0, bl0, Wr0, Wl1, bl1, Wr1, Wl2, bl2, Wr2,
           W1, b1, W2, b2):
    src = edge_index[0]
    dst = edge_index[1]
    pad = EPAD - E
    ar = jnp.arange(pad, dtype=jnp.int32)
    srcp = jnp.concatenate([src, ar % N]).reshape(EPAD // CHUNK, CHUNK)
    dstp = jnp.concatenate([dst, N + ar % (NPAD - N)]).reshape(
        EPAD // CHUNK, CHUNK)
    zs = jnp.zeros((RPT, D), jnp.float32)
    zc = jnp.zeros((RPT,), jnp.float32)
    ones = jnp.ones((CHUNK,), jnp.float32)
    psum, cnt0, cnt1 = _make_seg_call()(srcp, dstp, x, zs, zc, ones)
    return _make_head_call()(x, psum, cnt0[:, None], cnt1[:, None], Wl2, bl2.reshape(1, H), Wr2,
                             W1, b1.reshape(1, H), W2, b2.reshape(1, C))


# speedup vs baseline: 3.9188x; 1.0102x over previous
import functools

import jax
import jax.numpy as jnp
from jax import lax
from jax.experimental import pallas as pl
from jax.experimental.pallas import tpu as pltpu
from jax.experimental.pallas import tpu_sc as plsc

N = 10000
E = 320000
D = 128
H = 128
C = 64

NC = 2
NS = 16
NW = NC * NS

CHUNK = 128
CPW = 80
HPW = 40
EPAD = NW * CPW * CHUNK
NPAD = 10240
RPT = NPAD // NS


def _seg_body(src_hbm, dst_hbm, x_hbm, zs_hbm, zc_hbm, ones_hbm,
              osum_hbm, ocnt0_hbm, ocnt1_hbm,
              acc_sum, acc_cnt, src_v, dst_v, rows0_v, rows1_v, ones_v,
              sem0, sem1, semc):
    cid = lax.axis_index("c")
    sid = lax.axis_index("s")
    w = sid * NC + cid

    czs = pltpu.async_copy(zs_hbm, acc_sum.at[pl.ds(sid * RPT, RPT)], sem0)
    czc = pltpu.async_copy(zc_hbm, acc_cnt.at[pl.ds(sid * RPT, RPT)], sem1)
    pltpu.sync_copy(ones_hbm, ones_v)
    czs.wait()
    czc.wait()
    plsc.subcore_barrier()

    for h in range(2):
        pltpu.sync_copy(src_hbm.at[pl.ds(w * CPW + h * HPW, HPW)], src_v)
        pltpu.sync_copy(dst_hbm.at[pl.ds(w * CPW + h * HPW, HPW)], dst_v)
        pltpu.async_copy(x_hbm.at[src_v.at[0]], rows0_v, sem0)

        @pl.loop(0, HPW, step=2)
        def _(j0):
            j1 = j0 + 1
            pltpu.async_copy(x_hbm.at[src_v.at[j1]], rows1_v, sem1)
            pltpu.make_async_copy(x_hbm.at[src_v.at[j0]], rows0_v, sem0).wait()
            pltpu.sync_copy(rows0_v, acc_sum.at[dst_v.at[j0]], add=True)
            pltpu.async_copy(ones_v, acc_cnt.at[dst_v.at[j0]], semc, add=True)

            @pl.when(j0 + 2 < HPW)
            def _():
                pltpu.async_copy(x_hbm.at[src_v.at[j0 + 2]], rows0_v, sem0)

            pltpu.make_async_copy(x_hbm.at[src_v.at[j1]], rows1_v, sem1).wait()
            pltpu.sync_copy(rows1_v, acc_sum.at[dst_v.at[j1]], add=True)
            pltpu.async_copy(ones_v, acc_cnt.at[dst_v.at[j1]], semc, add=True)

        @pl.loop(0, HPW)
        def _(j):
            pltpu.make_async_copy(ones_v, acc_cnt.at[dst_v.at[0]], semc).wait()

    plsc.subcore_barrier()
    co = pltpu.async_copy(acc_sum.at[pl.ds(sid * RPT, RPT)],
                          osum_hbm.at[cid, pl.ds(sid * RPT, RPT)], sem0)

    @pl.when(cid == 0)
    def _():
        pltpu.sync_copy(acc_cnt.at[pl.ds(sid * RPT, RPT)],
                        ocnt0_hbm.at[pl.ds(sid * RPT, RPT)])

    @pl.when(cid == 1)
    def _():
        pltpu.sync_copy(acc_cnt.at[pl.ds(sid * RPT, RPT)],
                        ocnt1_hbm.at[pl.ds(sid * RPT, RPT)])
    co.wait()


@functools.cache
def _make_seg_call():
    return pl.kernel(
        _seg_body,
        out_type=[
            jax.ShapeDtypeStruct((NC, NPAD, D), jnp.float32),
            jax.ShapeDtypeStruct((NPAD,), jnp.float32),
            jax.ShapeDtypeStruct((NPAD,), jnp.float32),
        ],
        mesh=plsc.VectorSubcoreMesh(core_axis_name="c", subcore_axis_name="s",
                                    num_cores=NC, num_subcores=NS),
        scratch_types=[
            pltpu.VMEM_SHARED((NPAD, D), jnp.float32),
            pltpu.VMEM_SHARED((NPAD,), jnp.float32),
            pltpu.VMEM((HPW, CHUNK), jnp.int32),
            pltpu.VMEM((HPW, CHUNK), jnp.int32),
            pltpu.VMEM((CHUNK, D), jnp.float32),
            pltpu.VMEM((CHUNK, D), jnp.float32),
            pltpu.VMEM((CHUNK,), jnp.float32),
            pltpu.SemaphoreType.DMA,
            pltpu.SemaphoreType.DMA,
            pltpu.SemaphoreType.DMA,
        ],
    )


BLK = 1000
NBLK = N // BLK


def _mmT(a, b):
    return lax.dot_general(a, b, (((1,), (1,)), ((), ())),
                           preferred_element_type=jnp.float32)


def _head_body(x_ref, ps_ref, pc0_ref, pc1_ref, wl_ref, bl_ref, wr_ref,
               w1_ref, b1_ref, w2_ref, b2_ref, o_ref, acc_ref):
    i = pl.program_id(0)

    @pl.when(i == 0)
    def _():
        acc_ref[...] = jnp.zeros_like(acc_ref)

    s = ps_ref[0] + ps_ref[1]
    cnt = pc0_ref[...] + pc1_ref[...]
    mean = s / jnp.maximum(cnt, 1.0)
    pre = _mmT(mean, wl_ref[...]) + bl_ref[...] + _mmT(x_ref[...], wr_ref[...])
    h = jnp.maximum(pre, 0.0)
    acc_ref[...] += jnp.sum(h, axis=0, keepdims=True)

    @pl.when(i == NBLK - 1)
    def _():
        pooled = acc_ref[...] * (1.0 / N)
        z = jnp.maximum(_mmT(pooled, w1_ref[...]) + b1_ref[...], 0.0)
        z2 = _mmT(z, w2_ref[...]) + b2_ref[...]
        m = jnp.max(z2, axis=1, keepdims=True)
        e = z2 - m
        lse = jnp.log(jnp.sum(jnp.exp(e), axis=1, keepdims=True))
        o_ref[...] = e - lse


@functools.cache
def _make_head_call(interpret: bool = False):
    return pl.pallas_call(
        _head_body,
        grid=(NBLK,),
        in_specs=[
            pl.BlockSpec((BLK, D), lambda i: (i, 0)),
            pl.BlockSpec((NC, BLK, D), lambda i: (0, i, 0)),
            pl.BlockSpec((BLK, 1), lambda i: (i, 0)),
            pl.BlockSpec((BLK, 1), lambda i: (i, 0)),
            pl.BlockSpec((H, D), lambda i: (0, 0)),
            pl.BlockSpec((1, H), lambda i: (0, 0)),
            pl.BlockSpec((H, D), lambda i: (0, 0)),
            pl.BlockSpec((H, H), lambda i: (0, 0)),
            pl.BlockSpec((1, H), lambda i: (0, 0)),
            pl.BlockSpec((C, H), lambda i: (0, 0)),
            pl.BlockSpec((1, C), lambda i: (0, 0)),
        ],
        out_specs=pl.BlockSpec((1, C), lambda i: (0, 0)),
        out_shape=jax.ShapeDtypeStruct((1, C), jnp.float32),
        scratch_shapes=[pltpu.VMEM((1, H), jnp.float32)],
        interpret=interpret,
    )


def kernel(x, edge_index, Wl0, bl0, Wr0, Wl1, bl1, Wr1, Wl2, bl2, Wr2,
           W1, b1, W2, b2):
    src = edge_index[0]
    dst = edge_index[1]
    pad = EPAD - E
    ar = jnp.arange(pad, dtype=jnp.int32)
    srcp = jnp.concatenate([src, ar % N]).reshape(EPAD // CHUNK, CHUNK)
    dstp = jnp.concatenate([dst, N + ar % (NPAD - N)]).reshape(
        EPAD // CHUNK, CHUNK)
    zs = jnp.zeros((RPT, D), jnp.float32)
    zc = jnp.zeros((RPT,), jnp.float32)
    ones = jnp.ones((CHUNK,), jnp.float32)
    psum, cnt0, cnt1 = _make_seg_call()(srcp, dstp, x, zs, zc, ones)
    return _make_head_call()(x, psum, cnt0[:, None], cnt1[:, None], Wl2, bl2.reshape(1, H), Wr2,
                             W1, b1.reshape(1, H), W2, b2.reshape(1, C))
